# baseline (speedup 1.0000x reference)
import functools

import jax
import jax.numpy as jnp
from jax import lax
from jax.experimental import pallas as pl
from jax.experimental.pallas import tpu as pltpu
from jax.experimental.pallas import tpu_sc as plsc

_K = 2


def _sc_worker_id():
    info = plsc.get_sparse_core_info()
    return lax.axis_index("s") * info.num_cores + lax.axis_index("c")


def _num_workers():
    info = plsc.get_sparse_core_info()
    return info.num_cores * info.num_subcores


def _route_body(x_ref, wg_ref, bg_ref, pp0_ref, pp1_ref, g0_ref, g1_ref,
                blk_ref, *, blk_b, n_g, n_pad):
    t = x_ref.shape[0]
    e = wg_ref.shape[1]
    logits = jnp.dot(x_ref[...], wg_ref[...],
                     preferred_element_type=jnp.float32) + bg_ref[0]
    ei = lax.broadcasted_iota(jnp.int32, (t, e), 1)
    v1 = jnp.max(logits, axis=1)
    i1 = jnp.min(jnp.where(logits == v1[:, None], ei, e), axis=1)
    masked = jnp.where(ei == i1[:, None], -jnp.inf, logits)
    v2 = jnp.max(masked, axis=1)
    i2 = jnp.min(jnp.where(masked == v2[:, None], ei, e), axis=1)
    z = jnp.exp(v2 - v1)
    gate0 = 1.0 / (1.0 + z)
    gate1 = 1.0 - gate0

    oh0 = (ei == i1[:, None]).astype(jnp.float32)
    oh1 = (ei == i2[:, None]).astype(jnp.float32)

    def excl_cumsum(oh, cs=256):
        cs = min(cs, t)
        lt = (lax.broadcasted_iota(jnp.int32, (cs, cs), 0)
              > lax.broadcasted_iota(jnp.int32, (cs, cs), 1)).astype(jnp.float32)
        pieces = []
        carry = jnp.zeros((1, e), jnp.float32)
        for c in range(t // cs):
            blkv = oh[c * cs:(c + 1) * cs, :]
            within = jnp.dot(lt, blkv, preferred_element_type=jnp.float32)
            pieces.append(within + carry)
            carry = carry + jnp.sum(blkv, axis=0, keepdims=True)
        return jnp.concatenate(pieces, axis=0), carry

    rank0, tot0 = excl_cumsum(oh0)
    rank1_, tot1 = excl_cumsum(oh1)
    rank1 = rank1_ + tot0
    counts = tot0 + tot1
    nb = jnp.floor((counts + (blk_b - 1)) / blk_b)
    ut = (lax.broadcasted_iota(jnp.int32, (e, e), 0)
          <= lax.broadcasted_iota(jnp.int32, (e, e), 1)).astype(jnp.float32)
    nb_csum = jnp.dot(nb, ut, preferred_element_type=jnp.float32)
    first_blk = nb_csum - nb

    fb0 = jnp.sum(oh0 * first_blk, axis=1)
    fb1 = jnp.sum(oh1 * first_blk, axis=1)
    r0 = jnp.sum(oh0 * rank0, axis=1)
    r1 = jnp.sum(oh1 * rank1, axis=1)
    pp0_ref[...] = (fb0 * blk_b + r0)[:, None].astype(jnp.int32)
    pp1_ref[...] = (fb1 * blk_b + r1)[:, None].astype(jnp.int32)
    g0_ref[...] = jnp.broadcast_to(gate0[:, None], (t, 16))
    g1_ref[...] = jnp.broadcast_to(gate1[:, None], (t, 16))

    gi = lax.broadcasted_iota(jnp.int32, (1, n_pad), 1).astype(jnp.float32)
    blk_e = jnp.sum((gi[:, :, None] >= nb_csum[:, None, :]).astype(jnp.float32),
                    axis=2)
    blk_e = jnp.minimum(blk_e, e - 1)
    total = jnp.broadcast_to(nb_csum[:, -1:], (1, n_pad))
    is_tot = lax.broadcasted_iota(jnp.int32, (1, n_pad), 1) == n_g
    blk_ref[...] = jnp.where(is_tot, total, blk_e).astype(jnp.int32)


def _make_scatter(t, d, s_tot):
    nw = _num_workers()
    tok_per_w = t // nw
    chunk = tok_per_w
    while (chunk * d * 4) > (380 * 1024 // 2):
        chunk //= 2
    n_chunks = tok_per_w // chunk
    mesh = plsc.VectorSubcoreMesh(core_axis_name="c", subcore_axis_name="s")

    @functools.partial(
        pl.kernel, mesh=mesh,
        out_type=jax.ShapeDtypeStruct((s_tot, d), jnp.float32),
        scratch_types=[
            pltpu.VMEM((chunk,), jnp.int32),
            pltpu.VMEM((chunk,), jnp.int32),
            pltpu.VMEM((chunk, d), jnp.float32),
            pltpu.SemaphoreType.DMA,
            pltpu.SemaphoreType.DMA,
        ],
    )
    def scat(x_hbm, p0_hbm, p1_hbm, out_hbm, idx0, idx1, buf, sem0, sem1):
        wid = _sc_worker_id()
        base = wid * tok_per_w
        for c in range(n_chunks):
            off = base + c * chunk
            pltpu.sync_copy(x_hbm.at[pl.ds(off, chunk)], buf)
            pltpu.sync_copy(p0_hbm.at[pl.ds(off, chunk)], idx0)
            pltpu.sync_copy(p1_hbm.at[pl.ds(off, chunk)], idx1)
            cp0 = pltpu.async_copy(buf, out_hbm.at[idx0], sem0)
            cp1 = pltpu.async_copy(buf, out_hbm.at[idx1], sem1)
            cp0.wait()
            cp1.wait()

    return scat


def _make_combine(t, d, s_tot):
    nw = _num_workers()
    tok_per_w = t // nw
    chunk = tok_per_w
    while (chunk * d * 4) > (380 * 1024 // 2):
        chunk //= 2
    n_chunks = tok_per_w // chunk
    mesh = plsc.VectorSubcoreMesh(core_axis_name="c", subcore_axis_name="s")

    @functools.partial(
        pl.kernel, mesh=mesh,
        out_type=jax.ShapeDtypeStruct((t, d), jnp.float32),
        scratch_types=[
            pltpu.VMEM((chunk,), jnp.int32),
            pltpu.VMEM((chunk,), jnp.int32),
            pltpu.VMEM((chunk, 16), jnp.float32),
            pltpu.VMEM((chunk, 16), jnp.float32),
            pltpu.VMEM((chunk, d), jnp.float32),
            pltpu.VMEM((chunk, d), jnp.float32),
            pltpu.SemaphoreType.DMA,
            pltpu.SemaphoreType.DMA,
        ],
    )
    def comb(y_hbm, p0_hbm, p1_hbm, g0_hbm, g1_hbm, out_hbm,
             idx0, idx1, g0v, g1v, buf0, buf1, sem0, sem1):
        wid = _sc_worker_id()
        base = wid * tok_per_w
        for c in range(n_chunks):
            off = base + c * chunk
            pltpu.sync_copy(p0_hbm.at[pl.ds(off, chunk)], idx0)
            pltpu.sync_copy(p1_hbm.at[pl.ds(off, chunk)], idx1)
            pltpu.sync_copy(g0_hbm.at[pl.ds(off, chunk)], g0v)
            pltpu.sync_copy(g1_hbm.at[pl.ds(off, chunk)], g1v)
            cp0 = pltpu.async_copy(y_hbm.at[idx0], buf0, sem0)
            cp1 = pltpu.async_copy(y_hbm.at[idx1], buf1, sem1)
            cp0.wait()
            cp1.wait()

            @pl.loop(0, chunk)
            def _(r):
                gv0 = g0v[r, :]
                gv1 = g1v[r, :]
                for j in range(d // 16):
                    sl = pl.ds(j * 16, 16)
                    buf0[r, sl] = buf0[r, sl] * gv0 + buf1[r, sl] * gv1

            pltpu.sync_copy(buf0, out_hbm.at[pl.ds(off, chunk)])

    return comb


def _ffn_body(blk_e_ref, xin_ref, w1_ref, b1_ref, w2_ref, b2_ref, y_ref,
              *, n_g):
    g = pl.program_id(0)

    @pl.when(g < blk_e_ref[n_g])
    def _():
        xb = xin_ref[...].astype(jnp.bfloat16)
        h = jnp.dot(xb, w1_ref[0], preferred_element_type=jnp.float32)
        h = jnp.maximum(h + b1_ref[0], 0.0)
        y = jnp.dot(h.astype(jnp.bfloat16), w2_ref[0],
                    preferred_element_type=jnp.float32)
        y_ref[...] = y + b2_ref[0]


def kernel(moe_inp, Wg, bg, w1, b1, w2, b2):
    x = moe_inp
    t, d = x.shape
    e, _, dff = w1.shape
    k = _K
    tk = t * k
    blk_b = min(128, tk)
    n_g = tk // blk_b + e
    s_tot = n_g * blk_b
    n_pad = 64
    assert n_g < n_pad

    pp0, pp1, g0b, g1b, blk = pl.pallas_call(
        functools.partial(_route_body, blk_b=blk_b, n_g=n_g, n_pad=n_pad),
        out_shape=[
            jax.ShapeDtypeStruct((t, 1), jnp.int32),
            jax.ShapeDtypeStruct((t, 1), jnp.int32),
            jax.ShapeDtypeStruct((t, 16), jnp.float32),
            jax.ShapeDtypeStruct((t, 16), jnp.float32),
            jax.ShapeDtypeStruct((1, n_pad), jnp.int32),
        ],
    )(x, Wg, bg.reshape(1, e))
    pp0 = pp0.reshape(t)
    pp1 = pp1.reshape(t)
    blk = blk.reshape(n_pad)

    xin = _make_scatter(t, d, s_tot)(x, pp0, pp1)

    b1r = b1.reshape(e, 1, dff)
    b2r = b2.reshape(e, 1, d)
    w1_16 = w1.astype(jnp.bfloat16)
    w2_16 = w2.astype(jnp.bfloat16)

    grid_spec = pltpu.PrefetchScalarGridSpec(
        num_scalar_prefetch=1,
        grid=(n_g,),
        in_specs=[
            pl.BlockSpec((blk_b, d), lambda g, be: (g, 0)),
            pl.BlockSpec((1, d, dff), lambda g, be: (be[g], 0, 0)),
            pl.BlockSpec((1, 1, dff), lambda g, be: (be[g], 0, 0)),
            pl.BlockSpec((1, dff, d), lambda g, be: (be[g], 0, 0)),
            pl.BlockSpec((1, 1, d), lambda g, be: (be[g], 0, 0)),
        ],
        out_specs=pl.BlockSpec((blk_b, d), lambda g, be: (g, 0)),
    )
    y = pl.pallas_call(
        functools.partial(_ffn_body, n_g=n_g),
        grid_spec=grid_spec,
        out_shape=jax.ShapeDtypeStruct((s_tot, d), jnp.float32),
        compiler_params=pltpu.CompilerParams(
            dimension_semantics=("arbitrary",),
        ),
    )(blk, xin, w1_16, b1r, w2_16, b2r)

    return _make_combine(t, d, s_tot)(y, pp0, pp1, g0b, g1b)


# speedup vs baseline: 1.0327x; 1.0327x over previous
import functools

import jax
import jax.numpy as jnp
from jax import lax
from jax.experimental import pallas as pl
from jax.experimental.pallas import tpu as pltpu
from jax.experimental.pallas import tpu_sc as plsc

_K = 2


def _sc_worker_id():
    info = plsc.get_sparse_core_info()
    return lax.axis_index("s") * info.num_cores + lax.axis_index("c")


def _num_workers():
    info = plsc.get_sparse_core_info()
    return info.num_cores * info.num_subcores


def _route_body(x_ref, wg_ref, bg_ref, pp0_ref, pp1_ref, g0_ref, g1_ref,
                blk_ref, *, blk_b, n_g, n_pad):
    t = x_ref.shape[0]
    e = wg_ref.shape[1]
    logits = jnp.dot(x_ref[...], wg_ref[...],
                     preferred_element_type=jnp.float32) + bg_ref[0]
    ei = lax.broadcasted_iota(jnp.int32, (t, e), 1)
    v1 = jnp.max(logits, axis=1)
    i1 = jnp.min(jnp.where(logits == v1[:, None], ei, e), axis=1)
    masked = jnp.where(ei == i1[:, None], -jnp.inf, logits)
    v2 = jnp.max(masked, axis=1)
    i2 = jnp.min(jnp.where(masked == v2[:, None], ei, e), axis=1)
    z = jnp.exp(v2 - v1)
    gate0 = 1.0 / (1.0 + z)
    gate1 = 1.0 - gate0

    oh0 = (ei == i1[:, None]).astype(jnp.float32)
    oh1 = (ei == i2[:, None]).astype(jnp.float32)

    def excl_cumsum(oh, cs=256):
        cs = min(cs, t)
        lt = (lax.broadcasted_iota(jnp.int32, (cs, cs), 0)
              > lax.broadcasted_iota(jnp.int32, (cs, cs), 1)).astype(jnp.float32)
        pieces = []
        carry = jnp.zeros((1, e), jnp.float32)
        for c in range(t // cs):
            blkv = oh[c * cs:(c + 1) * cs, :]
            within = jnp.dot(lt, blkv, preferred_element_type=jnp.float32)
            pieces.append(within + carry)
            carry = carry + jnp.sum(blkv, axis=0, keepdims=True)
        return jnp.concatenate(pieces, axis=0), carry

    rank0, tot0 = excl_cumsum(oh0)
    rank1_, tot1 = excl_cumsum(oh1)
    rank1 = rank1_ + tot0
    counts = tot0 + tot1
    nb = jnp.floor((counts + (blk_b - 1)) / blk_b)
    ut = (lax.broadcasted_iota(jnp.int32, (e, e), 0)
          <= lax.broadcasted_iota(jnp.int32, (e, e), 1)).astype(jnp.float32)
    nb_csum = jnp.dot(nb, ut, preferred_element_type=jnp.float32)
    first_blk = nb_csum - nb

    fb0 = jnp.sum(oh0 * first_blk, axis=1)
    fb1 = jnp.sum(oh1 * first_blk, axis=1)
    r0 = jnp.sum(oh0 * rank0, axis=1)
    r1 = jnp.sum(oh1 * rank1, axis=1)
    pp0_ref[...] = (fb0 * blk_b + r0)[:, None].astype(jnp.int32)
    pp1_ref[...] = (fb1 * blk_b + r1)[:, None].astype(jnp.int32)
    g0_ref[...] = jnp.broadcast_to(gate0[:, None], (t, 16))
    g1_ref[...] = jnp.broadcast_to(gate1[:, None], (t, 16))

    gi = lax.broadcasted_iota(jnp.int32, (1, n_pad), 1).astype(jnp.float32)
    blk_e = jnp.sum((gi[:, :, None] >= nb_csum[:, None, :]).astype(jnp.float32),
                    axis=2)
    blk_e = jnp.minimum(blk_e, e - 1)
    total = jnp.broadcast_to(nb_csum[:, -1:], (1, n_pad))
    is_tot = lax.broadcasted_iota(jnp.int32, (1, n_pad), 1) == n_g
    blk_ref[...] = jnp.where(is_tot, total, blk_e).astype(jnp.int32)


def _make_scatter(t, d, s_tot):
    nw = _num_workers()
    tok_per_w = t // nw
    chunk = tok_per_w
    while (chunk * d * 4) > (380 * 1024 // 2):
        chunk //= 2
    n_chunks = tok_per_w // chunk
    mesh = plsc.VectorSubcoreMesh(core_axis_name="c", subcore_axis_name="s")

    @functools.partial(
        pl.kernel, mesh=mesh,
        out_type=jax.ShapeDtypeStruct((s_tot, d), jnp.float32),
        scratch_types=[
            pltpu.VMEM((chunk,), jnp.int32),
            pltpu.VMEM((chunk,), jnp.int32),
            pltpu.VMEM((chunk, d), jnp.float32),
            pltpu.SemaphoreType.DMA,
            pltpu.SemaphoreType.DMA,
        ],
    )
    def scat(x_hbm, p0_hbm, p1_hbm, out_hbm, idx0, idx1, buf, sem0, sem1):
        wid = _sc_worker_id()
        base = wid * tok_per_w
        for c in range(n_chunks):
            off = base + c * chunk
            pltpu.sync_copy(x_hbm.at[pl.ds(off, chunk)], buf)
            pltpu.sync_copy(p0_hbm.at[pl.ds(off, chunk)], idx0)
            pltpu.sync_copy(p1_hbm.at[pl.ds(off, chunk)], idx1)
            cp0 = pltpu.async_copy(buf, out_hbm.at[idx0], sem0)
            cp1 = pltpu.async_copy(buf, out_hbm.at[idx1], sem1)
            cp0.wait()
            cp1.wait()

    return scat


def _make_combine(t, d, s_tot):
    nw = _num_workers()
    tok_per_w = t // nw
    chunk = tok_per_w
    while (chunk * d * 4) > (380 * 1024 // 2):
        chunk //= 2
    n_chunks = tok_per_w // chunk
    mesh = plsc.VectorSubcoreMesh(core_axis_name="c", subcore_axis_name="s")

    @functools.partial(
        pl.kernel, mesh=mesh,
        out_type=jax.ShapeDtypeStruct((t, d), jnp.float32),
        scratch_types=[
            pltpu.VMEM((chunk,), jnp.int32),
            pltpu.VMEM((chunk,), jnp.int32),
            pltpu.VMEM((chunk, 16), jnp.float32),
            pltpu.VMEM((chunk, 16), jnp.float32),
            pltpu.VMEM((chunk, d), jnp.float32),
            pltpu.VMEM((chunk, d), jnp.float32),
            pltpu.SemaphoreType.DMA,
            pltpu.SemaphoreType.DMA,
        ],
    )
    def comb(y_hbm, p0_hbm, p1_hbm, g0_hbm, g1_hbm, out_hbm,
             idx0, idx1, g0v, g1v, buf0, buf1, sem0, sem1):
        wid = _sc_worker_id()
        base = wid * tok_per_w
        for c in range(n_chunks):
            off = base + c * chunk
            pltpu.sync_copy(p0_hbm.at[pl.ds(off, chunk)], idx0)
            pltpu.sync_copy(p1_hbm.at[pl.ds(off, chunk)], idx1)
            pltpu.sync_copy(g0_hbm.at[pl.ds(off, chunk)], g0v)
            pltpu.sync_copy(g1_hbm.at[pl.ds(off, chunk)], g1v)
            cp0 = pltpu.async_copy(y_hbm.at[idx0], buf0, sem0)
            cp1 = pltpu.async_copy(y_hbm.at[idx1], buf1, sem1)
            cp0.wait()
            cp1.wait()

            @pl.loop(0, chunk)
            def _(r):
                gv0 = g0v[r, :]
                gv1 = g1v[r, :]
                for j in range(d // 16):
                    sl = pl.ds(j * 16, 16)
                    buf0[r, sl] = buf0[r, sl] * gv0 + buf1[r, sl] * gv1

            pltpu.sync_copy(buf0, out_hbm.at[pl.ds(off, chunk)])

    return comb


def _ffn_body(blk_e_ref, xin_ref, w1_ref, b1_ref, w2_ref, b2_ref, y_ref,
              *, n_g):
    g = pl.program_id(0)

    @pl.when(g < blk_e_ref[n_g])
    def _():
        xb = xin_ref[...].astype(jnp.bfloat16)
        h = jnp.dot(xb, w1_ref[0], preferred_element_type=jnp.float32)
        h = jnp.maximum(h + b1_ref[0], 0.0)
        y = jnp.dot(h.astype(jnp.bfloat16), w2_ref[0],
                    preferred_element_type=jnp.float32)
        y_ref[...] = y + b2_ref[0]


def kernel(moe_inp, Wg, bg, w1, b1, w2, b2):
    x = moe_inp
    t, d = x.shape
    e, _, dff = w1.shape
    k = _K
    tk = t * k
    blk_b = min(256, tk)
    n_g = tk // blk_b + e
    s_tot = n_g * blk_b
    n_pad = 64
    assert n_g < n_pad

    pp0, pp1, g0b, g1b, blk = pl.pallas_call(
        functools.partial(_route_body, blk_b=blk_b, n_g=n_g, n_pad=n_pad),
        out_shape=[
            jax.ShapeDtypeStruct((t, 1), jnp.int32),
            jax.ShapeDtypeStruct((t, 1), jnp.int32),
            jax.ShapeDtypeStruct((t, 16), jnp.float32),
            jax.ShapeDtypeStruct((t, 16), jnp.float32),
            jax.ShapeDtypeStruct((1, n_pad), jnp.int32),
        ],
    )(x, Wg, bg.reshape(1, e))
    pp0 = pp0.reshape(t)
    pp1 = pp1.reshape(t)
    blk = blk.reshape(n_pad)

    xin = _make_scatter(t, d, s_tot)(x, pp0, pp1)

    b1r = b1.reshape(e, 1, dff)
    b2r = b2.reshape(e, 1, d)
    w1_16 = w1.astype(jnp.bfloat16)
    w2_16 = w2.astype(jnp.bfloat16)

    grid_spec = pltpu.PrefetchScalarGridSpec(
        num_scalar_prefetch=1,
        grid=(n_g,),
        in_specs=[
            pl.BlockSpec((blk_b, d), lambda g, be: (g, 0)),
            pl.BlockSpec((1, d, dff), lambda g, be: (be[g], 0, 0)),
            pl.BlockSpec((1, 1, dff), lambda g, be: (be[g], 0, 0)),
            pl.BlockSpec((1, dff, d), lambda g, be: (be[g], 0, 0)),
            pl.BlockSpec((1, 1, d), lambda g, be: (be[g], 0, 0)),
        ],
        out_specs=pl.BlockSpec((blk_b, d), lambda g, be: (g, 0)),
    )
    y = pl.pallas_call(
        functools.partial(_ffn_body, n_g=n_g),
        grid_spec=grid_spec,
        out_shape=jax.ShapeDtypeStruct((s_tot, d), jnp.float32),
        compiler_params=pltpu.CompilerParams(
            dimension_semantics=("arbitrary",),
        ),
    )(blk, xin, w1_16, b1r, w2_16, b2r)

    return _make_combine(t, d, s_tot)(y, pp0, pp1, g0b, g1b)


# speedup vs baseline: 1.0395x; 1.0066x over previous
import functools

import jax
import jax.numpy as jnp
from jax import lax
from jax.experimental import pallas as pl
from jax.experimental.pallas import tpu as pltpu
from jax.experimental.pallas import tpu_sc as plsc

_K = 2


def _sc_worker_id():
    info = plsc.get_sparse_core_info()
    return lax.axis_index("s") * info.num_cores + lax.axis_index("c")


def _num_workers():
    info = plsc.get_sparse_core_info()
    return info.num_cores * info.num_subcores


def _route_body(x_ref, wg_ref, bg_ref, pp0_ref, pp1_ref, g0_ref, g1_ref,
                blk_ref, *, blk_b, n_g, n_pad):
    t = x_ref.shape[0]
    e = wg_ref.shape[1]
    logits = jnp.dot(x_ref[...], wg_ref[...],
                     preferred_element_type=jnp.float32) + bg_ref[0]
    ei = lax.broadcasted_iota(jnp.int32, (t, e), 1)
    v1 = jnp.max(logits, axis=1)
    i1 = jnp.min(jnp.where(logits == v1[:, None], ei, e), axis=1)
    masked = jnp.where(ei == i1[:, None], -jnp.inf, logits)
    v2 = jnp.max(masked, axis=1)
    i2 = jnp.min(jnp.where(masked == v2[:, None], ei, e), axis=1)
    z = jnp.exp(v2 - v1)
    gate0 = 1.0 / (1.0 + z)
    gate1 = 1.0 - gate0

    oh0 = (ei == i1[:, None]).astype(jnp.float32)
    oh1 = (ei == i2[:, None]).astype(jnp.float32)

    def excl_cumsum(oh, cs=256):
        cs = min(cs, t)
        lt = (lax.broadcasted_iota(jnp.int32, (cs, cs), 0)
              > lax.broadcasted_iota(jnp.int32, (cs, cs), 1)).astype(jnp.float32)
        pieces = []
        carry = jnp.zeros((1, e), jnp.float32)
        for c in range(t // cs):
            blkv = oh[c * cs:(c + 1) * cs, :]
            within = jnp.dot(lt, blkv, preferred_element_type=jnp.float32)
            pieces.append(within + carry)
            carry = carry + jnp.sum(blkv, axis=0, keepdims=True)
        return jnp.concatenate(pieces, axis=0), carry

    rank0, tot0 = excl_cumsum(oh0)
    rank1_, tot1 = excl_cumsum(oh1)
    rank1 = rank1_ + tot0
    counts = tot0 + tot1
    nb = jnp.floor((counts + (blk_b - 1)) / blk_b)
    ut = (lax.broadcasted_iota(jnp.int32, (e, e), 0)
          <= lax.broadcasted_iota(jnp.int32, (e, e), 1)).astype(jnp.float32)
    nb_csum = jnp.dot(nb, ut, preferred_element_type=jnp.float32)
    first_blk = nb_csum - nb

    fb0 = jnp.sum(oh0 * first_blk, axis=1)
    fb1 = jnp.sum(oh1 * first_blk, axis=1)
    r0 = jnp.sum(oh0 * rank0, axis=1)
    r1 = jnp.sum(oh1 * rank1, axis=1)
    pp0_ref[...] = (fb0 * blk_b + r0)[:, None].astype(jnp.int32)
    pp1_ref[...] = (fb1 * blk_b + r1)[:, None].astype(jnp.int32)
    g0_ref[...] = jnp.broadcast_to(gate0[:, None], (t, 16))
    g1_ref[...] = jnp.broadcast_to(gate1[:, None], (t, 16))

    gi = lax.broadcasted_iota(jnp.int32, (1, n_pad), 1).astype(jnp.float32)
    blk_e = jnp.sum((gi[:, :, None] >= nb_csum[:, None, :]).astype(jnp.float32),
                    axis=2)
    blk_e = jnp.minimum(blk_e, e - 1)
    total = jnp.broadcast_to(nb_csum[:, -1:], (1, n_pad))
    is_tot = lax.broadcasted_iota(jnp.int32, (1, n_pad), 1) == n_g
    blk_ref[...] = jnp.where(is_tot, total, blk_e).astype(jnp.int32)


def _make_scatter(t, d, s_tot):
    nw = _num_workers()
    tok_per_w = t // nw
    chunk = tok_per_w
    while (chunk * d * 4) > (380 * 1024 // 2):
        chunk //= 2
    n_chunks = tok_per_w // chunk
    mesh = plsc.VectorSubcoreMesh(core_axis_name="c", subcore_axis_name="s")

    @functools.partial(
        pl.kernel, mesh=mesh,
        out_type=jax.ShapeDtypeStruct((s_tot, d), jnp.float32),
        scratch_types=[
            pltpu.VMEM((chunk,), jnp.int32),
            pltpu.VMEM((chunk,), jnp.int32),
            pltpu.VMEM((chunk,), jnp.int32),
            pltpu.VMEM((chunk,), jnp.int32),
            pltpu.VMEM((chunk, d), jnp.float32),
            pltpu.VMEM((chunk, d), jnp.float32),
            pltpu.SemaphoreType.DMA,
            pltpu.SemaphoreType.DMA,
            pltpu.SemaphoreType.DMA,
            pltpu.SemaphoreType.DMA,
        ],
    )
    def scat(x_hbm, p0_hbm, p1_hbm, out_hbm, idx0_a, idx1_a, idx0_b, idx1_b,
             buf_a, buf_b, sem_la, sem_lb, sem_s0, sem_s1):
        wid = _sc_worker_id()
        base = wid * tok_per_w
        bufs = (buf_a, buf_b)
        idx0s = (idx0_a, idx0_b)
        idx1s = (idx1_a, idx1_b)
        lsems = (sem_la, sem_lb)
        loads = [None] * n_chunks
        scats = [None] * n_chunks
        for c in range(min(2, n_chunks)):
            off = base + c * chunk
            loads[c] = pltpu.async_copy(
                x_hbm.at[pl.ds(off, chunk)], bufs[c % 2], lsems[c % 2])
            pltpu.sync_copy(p0_hbm.at[pl.ds(off, chunk)], idx0s[c % 2])
            pltpu.sync_copy(p1_hbm.at[pl.ds(off, chunk)], idx1s[c % 2])
        for c in range(n_chunks):
            b = c % 2
            loads[c].wait()
            scats[c] = (
                pltpu.async_copy(bufs[b], out_hbm.at[idx0s[b]], sem_s0),
                pltpu.async_copy(bufs[b], out_hbm.at[idx1s[b]], sem_s1),
            )
            nxt = c + 2
            if nxt < n_chunks:
                scats[c][0].wait()
                scats[c][1].wait()
                off = base + nxt * chunk
                loads[nxt] = pltpu.async_copy(
                    x_hbm.at[pl.ds(off, chunk)], bufs[b], lsems[b])
                pltpu.sync_copy(p0_hbm.at[pl.ds(off, chunk)], idx0s[b])
                pltpu.sync_copy(p1_hbm.at[pl.ds(off, chunk)], idx1s[b])
        for c in range(max(0, n_chunks - 2), n_chunks):
            scats[c][0].wait()
            scats[c][1].wait()

    return scat


def _make_combine(t, d, s_tot):
    nw = _num_workers()
    tok_per_w = t // nw
    chunk = tok_per_w
    while (chunk * d * 4 * 4) > (420 * 1024):
        chunk //= 2
    n_chunks = tok_per_w // chunk
    mesh = plsc.VectorSubcoreMesh(core_axis_name="c", subcore_axis_name="s")

    scratch = []
    for _ in range(2):
        scratch += [
            pltpu.VMEM((chunk,), jnp.int32),
            pltpu.VMEM((chunk,), jnp.int32),
            pltpu.VMEM((chunk, 16), jnp.float32),
            pltpu.VMEM((chunk, 16), jnp.float32),
            pltpu.VMEM((chunk, d), jnp.float32),
            pltpu.VMEM((chunk, d), jnp.float32),
            pltpu.SemaphoreType.DMA,
            pltpu.SemaphoreType.DMA,
            pltpu.SemaphoreType.DMA,
        ]

    @functools.partial(
        pl.kernel, mesh=mesh,
        out_type=jax.ShapeDtypeStruct((t, d), jnp.float32),
        scratch_types=scratch,
    )
    def comb(y_hbm, p0_hbm, p1_hbm, g0_hbm, g1_hbm, out_hbm, *scr):
        wid = _sc_worker_id()
        base = wid * tok_per_w
        sets = (scr[:9], scr[9:])
        gathers = [None] * n_chunks
        outs = [None] * n_chunks

        def start(c):
            idx0, idx1, g0v, g1v, buf0, buf1, sg0, sg1, _ = sets[c % 2]
            off = base + c * chunk
            pltpu.sync_copy(p0_hbm.at[pl.ds(off, chunk)], idx0)
            pltpu.sync_copy(p1_hbm.at[pl.ds(off, chunk)], idx1)
            pltpu.sync_copy(g0_hbm.at[pl.ds(off, chunk)], g0v)
            pltpu.sync_copy(g1_hbm.at[pl.ds(off, chunk)], g1v)
            gathers[c] = (pltpu.async_copy(y_hbm.at[idx0], buf0, sg0),
                          pltpu.async_copy(y_hbm.at[idx1], buf1, sg1))

        for c in range(min(2, n_chunks)):
            start(c)
        for c in range(n_chunks):
            idx0, idx1, g0v, g1v, buf0, buf1, sg0, sg1, so = sets[c % 2]
            gathers[c][0].wait()
            gathers[c][1].wait()

            @pl.loop(0, chunk)
            def _(r):
                gv0 = g0v[r, :]
                gv1 = g1v[r, :]
                for j in range(d // 16):
                    sl = pl.ds(j * 16, 16)
                    buf0[r, sl] = buf0[r, sl] * gv0 + buf1[r, sl] * gv1

            off = base + c * chunk
            outs[c] = pltpu.async_copy(buf0, out_hbm.at[pl.ds(off, chunk)], so)
            nxt = c + 2
            if nxt < n_chunks:
                outs[c].wait()
                start(nxt)
        for c in range(max(0, n_chunks - 2), n_chunks):
            outs[c].wait()

    return comb


def _ffn_body(blk_e_ref, xin_ref, w1_ref, b1_ref, w2_ref, b2_ref, y_ref,
              *, n_g):
    g = pl.program_id(0)

    @pl.when(g < blk_e_ref[n_g])
    def _():
        xb = xin_ref[...].astype(jnp.bfloat16)
        h = jnp.dot(xb, w1_ref[0], preferred_element_type=jnp.float32)
        h = jnp.maximum(h + b1_ref[0], 0.0)
        y = jnp.dot(h.astype(jnp.bfloat16), w2_ref[0],
                    preferred_element_type=jnp.float32)
        y_ref[...] = y + b2_ref[0]


def kernel(moe_inp, Wg, bg, w1, b1, w2, b2):
    x = moe_inp
    t, d = x.shape
    e, _, dff = w1.shape
    k = _K
    tk = t * k
    blk_b = min(256, tk)
    n_g = tk // blk_b + e
    s_tot = n_g * blk_b
    n_pad = 64
    assert n_g < n_pad

    pp0, pp1, g0b, g1b, blk = pl.pallas_call(
        functools.partial(_route_body, blk_b=blk_b, n_g=n_g, n_pad=n_pad),
        out_shape=[
            jax.ShapeDtypeStruct((t, 1), jnp.int32),
            jax.ShapeDtypeStruct((t, 1), jnp.int32),
            jax.ShapeDtypeStruct((t, 16), jnp.float32),
            jax.ShapeDtypeStruct((t, 16), jnp.float32),
            jax.ShapeDtypeStruct((1, n_pad), jnp.int32),
        ],
    )(x, Wg, bg.reshape(1, e))
    pp0 = pp0.reshape(t)
    pp1 = pp1.reshape(t)
    blk = blk.reshape(n_pad)

    xin = _make_scatter(t, d, s_tot)(x, pp0, pp1)

    b1r = b1.reshape(e, 1, dff)
    b2r = b2.reshape(e, 1, d)
    w1_16 = w1.astype(jnp.bfloat16)
    w2_16 = w2.astype(jnp.bfloat16)

    grid_spec = pltpu.PrefetchScalarGridSpec(
        num_scalar_prefetch=1,
        grid=(n_g,),
        in_specs=[
            pl.BlockSpec((blk_b, d), lambda g, be: (g, 0)),
            pl.BlockSpec((1, d, dff), lambda g, be: (be[g], 0, 0)),
            pl.BlockSpec((1, 1, dff), lambda g, be: (be[g], 0, 0)),
            pl.BlockSpec((1, dff, d), lambda g, be: (be[g], 0, 0)),
            pl.BlockSpec((1, 1, d), lambda g, be: (be[g], 0, 0)),
        ],
        out_specs=pl.BlockSpec((blk_b, d), lambda g, be: (g, 0)),
    )
    y = pl.pallas_call(
        functools.partial(_ffn_body, n_g=n_g),
        grid_spec=grid_spec,
        out_shape=jax.ShapeDtypeStruct((s_tot, d), jnp.float32),
        compiler_params=pltpu.CompilerParams(
            dimension_semantics=("arbitrary",),
        ),
    )(blk, xin, w1_16, b1r, w2_16, b2r)

    return _make_combine(t, d, s_tot)(y, pp0, pp1, g0b, g1b)
